# Initial kernel scaffold; baseline (speedup 1.0000x reference)
import jax
import jax.numpy as jnp
from jax.experimental import pallas as pl


def kernel(u_nchw, w_sq, w_ex, w_sse):
    raise NotImplementedError("write your pallas kernel here")


# speedup vs baseline: 1.2373x; 1.2373x over previous
import functools

import jax
import jax.numpy as jnp
from jax.experimental import pallas as pl
from jax.experimental.pallas import tpu as pltpu

_MIB = 1024 * 1024


def _scse_body(x_ref, wsq_t_ref, wex_t_ref, wsse_ref, out_ref, *, inv_hw):
    x = x_ref[...]
    nb = x.shape[0]

    m = jnp.sum(x, axis=2) * inv_hw
    s = jnp.dot(m, wsq_t_ref[...], preferred_element_type=jnp.float32)
    e = jnp.dot(s, wex_t_ref[...], preferred_element_type=jnp.float32)
    gate_c = jax.nn.sigmoid(e)

    w_row = wsse_ref[...]
    q = jnp.concatenate(
        [jnp.dot(w_row, x[i], preferred_element_type=jnp.float32)
         for i in range(nb)], axis=0)
    gate_s = jax.nn.sigmoid(q)

    out_ref[...] = x * (gate_c[:, :, None] + gate_s[:, None, :])


def _pick_batch_block(b, c, hw, itemsize):
    budget = 40 * _MIB
    bblk = 8
    while bblk > 1 and (b % bblk != 0 or 4 * bblk * c * hw * itemsize > budget):
        bblk //= 2
    return bblk


def kernel(u_nchw, w_sq, w_ex, w_sse):
    B, C, H, W = u_nchw.shape
    HW = H * W
    x = u_nchw.reshape(B, C, HW)
    bblk = _pick_batch_block(B, C, HW, x.dtype.itemsize)

    wsq_t = jnp.transpose(w_sq)
    wex_t = jnp.transpose(w_ex)
    wsse_row = w_sse.reshape(1, C)

    out = pl.pallas_call(
        functools.partial(_scse_body, inv_hw=1.0 / HW),
        out_shape=jax.ShapeDtypeStruct((B, C, HW), x.dtype),
        grid=(B // bblk,),
        in_specs=[
            pl.BlockSpec((bblk, C, HW), lambda b: (b, 0, 0)),
            pl.BlockSpec((C, C // 2), lambda b: (0, 0)),
            pl.BlockSpec((C // 2, C), lambda b: (0, 0)),
            pl.BlockSpec((1, C), lambda b: (0, 0)),
        ],
        out_specs=pl.BlockSpec((bblk, C, HW), lambda b: (b, 0, 0)),
        compiler_params=pltpu.CompilerParams(
            dimension_semantics=("parallel",),
            vmem_limit_bytes=56 * _MIB,
        ),
        cost_estimate=pl.CostEstimate(
            flops=6 * B * C * HW,
            transcendentals=B * (HW + C),
            bytes_accessed=2 * B * C * HW * x.dtype.itemsize,
        ),
    )(x, wsq_t, wex_t, wsse_row)
    return out.reshape(B, C, H, W)
